# baseline (speedup 1.0000x reference)
import jax
import jax.numpy as jnp
from jax import lax
from jax.experimental import pallas as pl
from jax.experimental.pallas import tpu as pltpu
from jax.experimental.pallas import tpu_sc as plsc

B, C, N = 8, 768, 1024
GROUP_SIZES = (96, 96, 192, 384)
FLATOFF = (0, 768, 1536, 3072)
NW = 32
CNT = tuple(8 * gs // NW for gs in GROUP_SIZES)


_CB = 128
_PB = 32


def _k1_body(q_ref, ms_ref, xm_s, cov_s, d2c_s, d2r_s):
    i = pl.program_id(0)
    ph = i // 6
    blk = i % 6
    i0 = blk * _CB

    @pl.when(ph == 0)
    def _xm():
        q = q_ref[...]
        cf = jnp.mean(q, axis=0)
        rm = jnp.mean(cf, axis=1, keepdims=True)
        xm_s[pl.ds(i0, _CB), :] = cf - rm

    @pl.when(ph == 1)
    def _cov():
        xmb = xm_s[pl.ds(i0, _CB), :]
        xm = xm_s[...]
        cov = lax.dot_general(xmb, xm, (((1,), (1,)), ((), ())),
                              preferred_element_type=jnp.float32) / (N - 1)
        cov_s[pl.ds(i0, _CB), :] = cov
        ri = i0 + lax.broadcasted_iota(jnp.int32, (_CB, C), 0)
        ci = lax.broadcasted_iota(jnp.int32, (_CB, C), 1)
        diag = jnp.where(ri == ci, cov, 0.0)
        d2c_s[pl.ds(i0, _CB), :] = jnp.sum(diag, axis=1, keepdims=True)
        part_r = jnp.sum(diag, axis=0, keepdims=True)
        prev = jnp.where(blk == 0, jnp.zeros_like(part_r), d2r_s[...])
        d2r_s[...] = prev + part_r

    @pl.when(ph == 2)
    def _ms():
        cov = cov_s[pl.ds(i0, _CB), :]
        dc = jnp.sqrt(d2c_s[pl.ds(i0, _CB), :])
        dr = jnp.sqrt(d2r_s[...])
        corr = cov / (dc * dr)
        ms_ref[...] = jnp.mean(corr, axis=1, keepdims=True)


_k1_call = pl.pallas_call(
    _k1_body,
    grid=(18,),
    in_specs=[pl.BlockSpec((B, _CB, N), lambda i: (0, jnp.minimum(i, 5), 0))],
    out_specs=pl.BlockSpec((_CB, 1),
                           lambda i: (jnp.where(i >= 12, i - 12, 0), 0)),
    out_shape=jax.ShapeDtypeStruct((C, 1), jnp.float32),
    scratch_shapes=[
        pltpu.VMEM((C, N), jnp.float32),
        pltpu.VMEM((C, C), jnp.float32),
        pltpu.VMEM((C, 1), jnp.float32),
        pltpu.VMEM((1, C), jnp.float32),
    ],
)


def _k2_body(msr_ref, msc_ref, idx2_ref, sidx_s):
    i = pl.program_id(0)

    @pl.when(i < 24)
    def _posinv():
        i0 = i * _PB
        mj = msr_ref[...]
        mi = msc_ref[pl.ds(i0, _PB), :]
        ri = i0 + lax.broadcasted_iota(jnp.int32, (_PB, C), 0)
        ci = lax.broadcasted_iota(jnp.int32, (_PB, C), 1)
        posmat = (mj > mi) | ((mj == mi) & (ci < ri))
        pos = jnp.sum(posmat.astype(jnp.int32), axis=1, keepdims=True)
        part = jnp.sum(jnp.where(pos == ci, ri, 0), axis=0, keepdims=True)
        prev = jnp.where(i == 0, jnp.zeros_like(part), sidx_s[...])
        sidx_s[...] = prev + part

    @pl.when(i == 24)
    def _emit():
        bi = lax.broadcasted_iota(jnp.int32, (B, C), 0) * C
        idx2_ref[...] = bi + sidx_s[...]


_k2_call = pl.pallas_call(
    _k2_body,
    grid=(25,),
    in_specs=[pl.BlockSpec((1, C), lambda i: (0, 0)),
              pl.BlockSpec((C, 1), lambda i: (0, 0))],
    out_specs=pl.BlockSpec((B, C), lambda i: (0, 0)),
    out_shape=jax.ShapeDtypeStruct((B, C), jnp.int32),
    scratch_shapes=[pltpu.VMEM((1, C), jnp.int32)],
)


def _stats_call(query):
    ms = _k1_call(query)
    idx2 = _k2_call(ms.reshape(1, C), ms)
    return idx2.reshape(B * C)


_CK = 24
_CHUNKS = tuple((g, off) for g in range(4) for off in range(0, CNT[g], _CK))
_IDX_OFF = (0, 24, 48, 96)
_OFFG = (0, 96, 192, 384)
_NB = 4


def _gather_body(idx_hbm, q_hbm, k_hbm, *rest):
    outs = rest[:8]
    idx_v = rest[8]
    bufs = rest[9:9 + _NB]
    gsems = rest[9 + _NB:9 + 2 * _NB]
    ssems = rest[9 + 2 * _NB:9 + 3 * _NB]
    w = lax.axis_index("s") * 2 + lax.axis_index("c")
    bb = w // 4
    for g in range(4):
        cnt = CNT[g]
        base = bb * C + _OFFG[g] + (w * cnt - bb * GROUP_SIZES[g])
        pltpu.sync_copy(idx_hbm.at[pl.ds(base, cnt)],
                        idx_v.at[pl.ds(_IDX_OFF[g], cnt)])
    srcs = (q_hbm, k_hbm)
    jobs = [(t,) + ch for t in range(2) for ch in _CHUNKS]
    n = len(jobs)
    copies = [None] * n
    scat = [None] * n

    def _start_scatter(c):
        t, g, off = jobs[c]
        scat[c] = pltpu.async_copy(
            bufs[c % _NB],
            outs[t * 4 + g].at[pl.ds(w * CNT[g] + off, _CK)],
            ssems[c % _NB])

    for c in range(n):
        t, g, off = jobs[c]
        b = c % _NB
        if c >= _NB:
            scat[c - _NB].wait()
        copies[c] = pltpu.async_copy(
            srcs[t].at[idx_v.at[pl.ds(_IDX_OFF[g] + off, _CK)]],
            bufs[b],
            gsems[b])
        if c >= 1:
            copies[c - 1].wait()
            _start_scatter(c - 1)
    copies[n - 1].wait()
    _start_scatter(n - 1)
    for c in range(n - _NB, n):
        scat[c].wait()


def _tc_copy_body(idx_ref, src_ref, *rest):
    outs = rest[:4]
    sem = rest[4]
    for g, gs in enumerate(GROUP_SIZES):
        def issue(j, _, g=g):
            s = idx_ref[0, _OFFG[g] + j]
            pltpu.make_async_copy(
                src_ref.at[:, pl.ds(s, 1), :],
                outs[g].at[:, pl.ds(j, 1), :],
                sem).start()
            return 0
        lax.fori_loop(0, gs, issue, 0)
    for g, gs in enumerate(GROUP_SIZES):
        def drain(j, _, g=g):
            pltpu.make_async_copy(
                src_ref.at[:, pl.ds(0, 1), :],
                outs[g].at[:, pl.ds(0, 1), :],
                sem).wait()
            return 0
        lax.fori_loop(0, gs, drain, 0)


_tc_copy_call = pl.pallas_call(
    _tc_copy_body,
    in_specs=[pl.BlockSpec(memory_space=pltpu.SMEM),
              pl.BlockSpec(memory_space=pl.ANY)],
    out_specs=tuple(pl.BlockSpec(memory_space=pl.ANY) for _ in range(4)),
    out_shape=tuple(jax.ShapeDtypeStruct((B, gs, N), jnp.float32)
                    for gs in GROUP_SIZES),
    scratch_shapes=[pltpu.SemaphoreType.DMA],
)


_gather_call_cache = []


def _gather_call(*args):
    if not _gather_call_cache:
        _gather_call_cache.append(pl.kernel(
            _gather_body,
            out_type=tuple(jax.ShapeDtypeStruct((8 * gs, N), jnp.float32)
                           for _ in range(2) for gs in GROUP_SIZES),
            mesh=plsc.VectorSubcoreMesh(core_axis_name="c",
                                        subcore_axis_name="s"),
            scratch_types=(
                [pltpu.VMEM((192,), jnp.int32)]
                + [pltpu.VMEM((_CK, N), jnp.float32) for _ in range(_NB)]
                + [pltpu.SemaphoreType.DMA for _ in range(2 * _NB)]
            ),
        ))
    return _gather_call_cache[0](*args)


def kernel(query, key, value):
    idx_flat = _stats_call(query)
    q2 = query.reshape(B * C, N)
    k2 = key.reshape(B * C, N)
    outs = _gather_call(idx_flat, q2, k2)
    sidx_row = idx_flat[:C].reshape(1, C)
    v_groups = _tc_copy_call(sidx_row, value)
    res = []
    for t in range(2):
        res.append(tuple(outs[t * 4 + g].reshape(B, GROUP_SIZES[g], N)
                         for g in range(4)))
    res.append(tuple(v_groups))
    return tuple(res)


# speedup vs baseline: 8.3949x; 8.3949x over previous
import jax
import jax.numpy as jnp
from jax import lax
from jax.experimental import pallas as pl
from jax.experimental.pallas import tpu as pltpu
from jax.experimental.pallas import tpu_sc as plsc

B, C, N = 8, 768, 1024
GROUP_SIZES = (96, 96, 192, 384)
FLATOFF = (0, 768, 1536, 3072)
NW = 32
CNT = tuple(8 * gs // NW for gs in GROUP_SIZES)


_CB = 128
_PB = 32


def _stats_body(q_ref, idx2_ref, xm_s, cov_s, d2c_s, d2r_s, msc_s, msr_s,
                sidx_s):
    i = pl.program_id(0)
    ph = jnp.minimum(i // 6, 3)
    blk = i % 6
    i0 = blk * _CB

    @pl.when(ph == 0)
    def _xm():
        q = q_ref[...]
        cf = jnp.mean(q, axis=0)
        rm = jnp.mean(cf, axis=1, keepdims=True)
        xm_s[pl.ds(i0, _CB), :] = cf - rm

    @pl.when(ph == 1)
    def _cov():
        xmb = xm_s[pl.ds(i0, _CB), :]
        xm = xm_s[...]
        cov = lax.dot_general(xmb, xm, (((1,), (1,)), ((), ())),
                              preferred_element_type=jnp.float32) / (N - 1)
        cov_s[pl.ds(i0, _CB), :] = cov
        ri = i0 + lax.broadcasted_iota(jnp.int32, (_CB, C), 0)
        ci = lax.broadcasted_iota(jnp.int32, (_CB, C), 1)
        diag = jnp.where(ri == ci, cov, 0.0)
        d2c_s[pl.ds(i0, _CB), :] = jnp.sum(diag, axis=1, keepdims=True)
        part_r = jnp.sum(diag, axis=0, keepdims=True)
        prev = jnp.where(blk == 0, jnp.zeros_like(part_r), d2r_s[...])
        d2r_s[...] = prev + part_r

    @pl.when(ph == 2)
    def _ms():
        cov = cov_s[pl.ds(i0, _CB), :]
        dc = jnp.sqrt(d2c_s[pl.ds(i0, _CB), :])
        dr = jnp.sqrt(d2r_s[...])
        corr = cov / (dc * dr)
        ms = jnp.mean(corr, axis=1, keepdims=True)
        msc_s[pl.ds(i0, _CB), :] = ms
        ri = i0 + lax.broadcasted_iota(jnp.int32, (_CB, C), 0)
        ci = lax.broadcasted_iota(jnp.int32, (_CB, C), 1)
        part = jnp.sum(jnp.where(ri == ci, ms, 0.0), axis=0, keepdims=True)
        prev = jnp.where(blk == 0, jnp.zeros_like(part), msr_s[...])
        msr_s[...] = prev + part

    @pl.when((ph == 3) & (i < 42))
    def _posinv():
        r0 = (i - 18) * _PB
        mj = msr_s[...]
        mi = msc_s[pl.ds(r0, _PB), :]
        ri = r0 + lax.broadcasted_iota(jnp.int32, (_PB, C), 0)
        ci = lax.broadcasted_iota(jnp.int32, (_PB, C), 1)
        posmat = (mj > mi) | ((mj == mi) & (ci < ri))
        pos = jnp.sum(posmat.astype(jnp.int32), axis=1, keepdims=True)
        part = jnp.sum(jnp.where(pos == ci, ri, 0), axis=0, keepdims=True)
        prev = jnp.where(i == 18, jnp.zeros_like(part), sidx_s[...])
        sidx_s[...] = prev + part

    @pl.when(i == 42)
    def _emit():
        bi = lax.broadcasted_iota(jnp.int32, (B, C), 0) * C
        idx2_ref[...] = bi + sidx_s[...]


_stats_kernel_call = pl.pallas_call(
    _stats_body,
    grid=(43,),
    in_specs=[pl.BlockSpec((B, _CB, N), lambda i: (0, jnp.minimum(i, 5), 0))],
    out_specs=pl.BlockSpec((B, C), lambda i: (0, 0)),
    out_shape=jax.ShapeDtypeStruct((B, C), jnp.int32),
    scratch_shapes=[
        pltpu.VMEM((C, N), jnp.float32),
        pltpu.VMEM((C, C), jnp.float32),
        pltpu.VMEM((C, 1), jnp.float32),
        pltpu.VMEM((1, C), jnp.float32),
        pltpu.VMEM((C, 1), jnp.float32),
        pltpu.VMEM((1, C), jnp.float32),
        pltpu.VMEM((1, C), jnp.int32),
    ],
)


def _stats_call(query):
    return _stats_kernel_call(query).reshape(B * C)


_CK = 24
_CHUNKS = tuple((g, off) for g in range(4) for off in range(0, CNT[g], _CK))
_IDX_OFF = (0, 24, 48, 96)
_OFFG = (0, 96, 192, 384)
_NB = 4


def _gather_body(idx_hbm, q_hbm, k_hbm, v_hbm, *rest):
    outs = rest[:12]
    idx_v = rest[12]
    bufs = rest[13:13 + _NB]
    gsems = rest[13 + _NB:13 + 2 * _NB]
    ssems = rest[13 + 2 * _NB:13 + 3 * _NB]
    w = lax.axis_index("s") * 2 + lax.axis_index("c")
    bb = w // 4
    for g in range(4):
        cnt = CNT[g]
        base = bb * C + _OFFG[g] + (w * cnt - bb * GROUP_SIZES[g])
        pltpu.sync_copy(idx_hbm.at[pl.ds(base, cnt)],
                        idx_v.at[pl.ds(_IDX_OFF[g], cnt)])
    srcs = (q_hbm, k_hbm, v_hbm)
    jobs = [(t,) + ch for t in range(3) for ch in _CHUNKS]
    n = len(jobs)
    copies = [None] * n
    scat = [None] * n

    def _start_scatter(c):
        t, g, off = jobs[c]
        scat[c] = pltpu.async_copy(
            bufs[c % _NB],
            outs[t * 4 + g].at[pl.ds(w * CNT[g] + off, _CK)],
            ssems[c % _NB])

    for c in range(n):
        t, g, off = jobs[c]
        b = c % _NB
        if c >= _NB:
            scat[c - _NB].wait()
        copies[c] = pltpu.async_copy(
            srcs[t].at[idx_v.at[pl.ds(_IDX_OFF[g] + off, _CK)]],
            bufs[b],
            gsems[b])
        if c >= 1:
            copies[c - 1].wait()
            _start_scatter(c - 1)
    copies[n - 1].wait()
    _start_scatter(n - 1)
    for c in range(n - _NB, n):
        scat[c].wait()


_gather_call_cache = []


def _gather_call(*args):
    if not _gather_call_cache:
        _gather_call_cache.append(pl.kernel(
            _gather_body,
            out_type=tuple(jax.ShapeDtypeStruct((8 * gs, N), jnp.float32)
                           for _ in range(3) for gs in GROUP_SIZES),
            mesh=plsc.VectorSubcoreMesh(core_axis_name="c",
                                        subcore_axis_name="s"),
            scratch_types=(
                [pltpu.VMEM((192,), jnp.int32)]
                + [pltpu.VMEM((_CK, N), jnp.float32) for _ in range(_NB)]
                + [pltpu.SemaphoreType.DMA for _ in range(2 * _NB)]
            ),
        ))
    return _gather_call_cache[0](*args)


def kernel(query, key, value):
    idx_flat = _stats_call(query)
    q2 = query.reshape(B * C, N)
    k2 = key.reshape(B * C, N)
    v2 = value.reshape(B * C, N)
    outs = _gather_call(idx_flat, q2, k2, v2)
    res = []
    for t in range(3):
        res.append(tuple(outs[t * 4 + g].reshape(B, GROUP_SIZES[g], N)
                         for g in range(4)))
    return tuple(res)


# speedup vs baseline: 8.5001x; 1.0125x over previous
import jax
import jax.numpy as jnp
from jax import lax
from jax.experimental import pallas as pl
from jax.experimental.pallas import tpu as pltpu
from jax.experimental.pallas import tpu_sc as plsc

B, C, N = 8, 768, 1024
GROUP_SIZES = (96, 96, 192, 384)
FLATOFF = (0, 768, 1536, 3072)
NW = 32
CNT = tuple(8 * gs // NW for gs in GROUP_SIZES)


_CB = 128
_PB = 32


def _stats_body(q_ref, idx2_ref, xm_s, cov_s, d2c_s, d2r_s, msc_s, msr_s,
                sidx_s):
    i = pl.program_id(0)
    ph = jnp.minimum(i // 6, 3)
    blk = i % 6
    i0 = blk * _CB

    @pl.when(ph == 0)
    def _xm():
        q = q_ref[...]
        cf = jnp.mean(q, axis=0)
        rm = jnp.mean(cf, axis=1, keepdims=True)
        xm_s[pl.ds(i0, _CB), :] = cf - rm

    @pl.when(ph == 1)
    def _cov():
        xmb = xm_s[pl.ds(i0, _CB), :]
        xm = xm_s[...]
        cov = lax.dot_general(xmb, xm, (((1,), (1,)), ((), ())),
                              preferred_element_type=jnp.float32) / (N - 1)
        cov_s[pl.ds(i0, _CB), :] = cov
        ri = i0 + lax.broadcasted_iota(jnp.int32, (_CB, C), 0)
        ci = lax.broadcasted_iota(jnp.int32, (_CB, C), 1)
        diag = jnp.where(ri == ci, cov, 0.0)
        d2c_s[pl.ds(i0, _CB), :] = jnp.sum(diag, axis=1, keepdims=True)
        part_r = jnp.sum(diag, axis=0, keepdims=True)
        prev = jnp.where(blk == 0, jnp.zeros_like(part_r), d2r_s[...])
        d2r_s[...] = prev + part_r

    @pl.when(ph == 2)
    def _ms():
        cov = cov_s[pl.ds(i0, _CB), :]
        dc = jnp.sqrt(d2c_s[pl.ds(i0, _CB), :])
        dr = jnp.sqrt(d2r_s[...])
        corr = cov / (dc * dr)
        ms = jnp.mean(corr, axis=1, keepdims=True)
        msc_s[pl.ds(i0, _CB), :] = ms
        ri = i0 + lax.broadcasted_iota(jnp.int32, (_CB, C), 0)
        ci = lax.broadcasted_iota(jnp.int32, (_CB, C), 1)
        part = jnp.sum(jnp.where(ri == ci, ms, 0.0), axis=0, keepdims=True)
        prev = jnp.where(blk == 0, jnp.zeros_like(part), msr_s[...])
        msr_s[...] = prev + part

    @pl.when((ph == 3) & (i < 42))
    def _posinv():
        r0 = (i - 18) * _PB
        mj = msr_s[...]
        mi = msc_s[pl.ds(r0, _PB), :]
        ri = r0 + lax.broadcasted_iota(jnp.int32, (_PB, C), 0)
        ci = lax.broadcasted_iota(jnp.int32, (_PB, C), 1)
        posmat = (mj > mi) | ((mj == mi) & (ci < ri))
        pos = jnp.sum(posmat.astype(jnp.int32), axis=1, keepdims=True)
        part = jnp.sum(jnp.where(pos == ci, ri, 0), axis=0, keepdims=True)
        prev = jnp.where(i == 18, jnp.zeros_like(part), sidx_s[...])
        sidx_s[...] = prev + part

    @pl.when(i == 42)
    def _emit():
        bi = lax.broadcasted_iota(jnp.int32, (B, C), 0) * C
        idx2_ref[...] = bi + sidx_s[...]


_stats_kernel_call = pl.pallas_call(
    _stats_body,
    grid=(43,),
    in_specs=[pl.BlockSpec((B, _CB, N), lambda i: (0, jnp.minimum(i, 5), 0))],
    out_specs=pl.BlockSpec((B, C), lambda i: (0, 0)),
    out_shape=jax.ShapeDtypeStruct((B, C), jnp.int32),
    scratch_shapes=[
        pltpu.VMEM((C, N), jnp.float32),
        pltpu.VMEM((C, C), jnp.float32),
        pltpu.VMEM((C, 1), jnp.float32),
        pltpu.VMEM((1, C), jnp.float32),
        pltpu.VMEM((C, 1), jnp.float32),
        pltpu.VMEM((1, C), jnp.float32),
        pltpu.VMEM((1, C), jnp.int32),
    ],
)


def _stats_call(query):
    return _stats_kernel_call(query).reshape(B * C)


_CK = 24
_CHUNKS = tuple((g, off) for g in range(4) for off in range(0, CNT[g], _CK))
_IDX_OFF = (0, 24, 48, 96)
_OFFG = (0, 96, 192, 384)
_NB = 4


def _gather_body(idx_hbm, q_hbm, k_hbm, v_hbm, *rest):
    outs = rest[:12]
    idx_v = rest[12]
    bufs = rest[13:13 + _NB]
    gsems = rest[13 + _NB:13 + 2 * _NB]
    ssems = rest[13 + 2 * _NB:13 + 3 * _NB]
    isems = rest[13 + 3 * _NB:13 + 3 * _NB + 4]
    w = lax.axis_index("s") * 2 + lax.axis_index("c")
    bb = w // 4
    icop = [None] * 4
    for g in range(4):
        cnt = CNT[g]
        base = bb * C + _OFFG[g] + (w * cnt - bb * GROUP_SIZES[g])
        icop[g] = pltpu.async_copy(idx_hbm.at[pl.ds(base, cnt)],
                                   idx_v.at[pl.ds(_IDX_OFF[g], cnt)],
                                   isems[g])
    srcs = (q_hbm, k_hbm, v_hbm)
    jobs = [(t,) + ch for t in range(3) for ch in _CHUNKS]
    n = len(jobs)
    copies = [None] * n
    scat = [None] * n

    def _start_scatter(c):
        t, g, off = jobs[c]
        scat[c] = pltpu.async_copy(
            bufs[c % _NB],
            outs[t * 4 + g].at[pl.ds(w * CNT[g] + off, _CK)],
            ssems[c % _NB])

    for c in range(n):
        t, g, off = jobs[c]
        b = c % _NB
        if t == 0 and off == 0:
            icop[g].wait()
        if c >= _NB:
            scat[c - _NB].wait()
        copies[c] = pltpu.async_copy(
            srcs[t].at[idx_v.at[pl.ds(_IDX_OFF[g] + off, _CK)]],
            bufs[b],
            gsems[b])
        if c >= 1:
            copies[c - 1].wait()
            _start_scatter(c - 1)
    copies[n - 1].wait()
    _start_scatter(n - 1)
    for c in range(n - _NB, n):
        scat[c].wait()


_gather_call_cache = []


def _gather_call(*args):
    if not _gather_call_cache:
        _gather_call_cache.append(pl.kernel(
            _gather_body,
            out_type=tuple(jax.ShapeDtypeStruct((8 * gs, N), jnp.float32)
                           for _ in range(3) for gs in GROUP_SIZES),
            mesh=plsc.VectorSubcoreMesh(core_axis_name="c",
                                        subcore_axis_name="s"),
            scratch_types=(
                [pltpu.VMEM((192,), jnp.int32)]
                + [pltpu.VMEM((_CK, N), jnp.float32) for _ in range(_NB)]
                + [pltpu.SemaphoreType.DMA for _ in range(2 * _NB + 4)]
            ),
        ))
    return _gather_call_cache[0](*args)


def kernel(query, key, value):
    idx_flat = _stats_call(query)
    q2 = query.reshape(B * C, N)
    k2 = key.reshape(B * C, N)
    v2 = value.reshape(B * C, N)
    outs = _gather_call(idx_flat, q2, k2, v2)
    res = []
    for t in range(3):
        res.append(tuple(outs[t * 4 + g].reshape(B, GROUP_SIZES[g], N)
                         for g in range(4)))
    return tuple(res)
